# baseline (speedup 1.0000x reference)
import functools

import jax
import jax.numpy as jnp
from jax import lax
from jax.experimental import pallas as pl
from jax.experimental.pallas import tpu as pltpu
from jax.experimental.pallas import tpu_sc as plsc

_N = 10000
_D = 128
_E = 320000
_NC = 2
_NS = 16
_NW = _NC * _NS
_CHUNK = 128
_NCHUNK = _E // _CHUNK
_CPS = 80
_CROWS = 2
_CH2 = _CROWS * _CHUNK
_NCH2 = _NW * _CPS // _CROWS
_NCHP = _NW * _CPS
_EPAD = _NCHP * _CHUNK - _E
_NACC = _N + 8
_RPS = 632
_RPS_LAST = _N - 15 * _RPS

_mesh = plsc.VectorSubcoreMesh(core_axis_name="c", subcore_axis_name="s")


@functools.partial(
    pl.kernel,
    mesh=_mesh,
    out_type=jax.ShapeDtypeStruct((_NC, _N, _D), jnp.float32),
    scratch_types=[
        pltpu.VMEM((_CHUNK,), jnp.int32),
        pltpu.VMEM((_CHUNK, _D), jnp.float32),
        pltpu.VMEM_SHARED((_NACC, _D), jnp.float32),
        pltpu.SemaphoreType.DMA,
    ],
)
def _sc_degree(dst_hbm, ones_hbm, zeros_hbm, out_hbm, di_v, ones_v, acc_sh, sem):
    cid = lax.axis_index("c")
    sid = lax.axis_index("s")
    wid = sid * _NC + cid
    row0 = sid * _RPS

    pltpu.sync_copy(ones_hbm, ones_v)

    @pl.when(sid < 15)
    def _():
        pltpu.sync_copy(zeros_hbm, acc_sh.at[pl.ds(row0, _RPS)])

    @pl.when(sid == 15)
    def _():
        pltpu.sync_copy(zeros_hbm.at[pl.ds(0, _RPS_LAST)],
                        acc_sh.at[pl.ds(row0, _RPS_LAST)])

    plsc.subcore_barrier()

    @pl.loop(wid, _NCHP, step=_NW)
    def _(c):
        pltpu.sync_copy(dst_hbm.at[pl.ds(c * _CHUNK, _CHUNK)], di_v)
        pltpu.sync_copy(ones_v, acc_sh.at[di_v], add=True)

    plsc.subcore_barrier()

    @pl.when(sid < 15)
    def _():
        pltpu.sync_copy(acc_sh.at[pl.ds(row0, _RPS)],
                        out_hbm.at[cid, pl.ds(row0, _RPS)])

    @pl.when(sid == 15)
    def _():
        pltpu.sync_copy(acc_sh.at[pl.ds(row0, _RPS_LAST)],
                        out_hbm.at[cid, pl.ds(row0, _RPS_LAST)])


@functools.partial(
    pl.kernel,
    mesh=_mesh,
    out_type=jax.ShapeDtypeStruct((_NC, _N, _D), jnp.float32),
    scratch_types=[
        pltpu.VMEM((_CH2,), jnp.int32),
        pltpu.VMEM((_CH2,), jnp.int32),
        pltpu.VMEM((_CH2, _D), jnp.float32),
        pltpu.VMEM_SHARED((_NACC, _D), jnp.float32),
        pltpu.SemaphoreType.DMA,
    ],
)
def _sc_aggregate(g_hbm, src_hbm, dst_hbm, zeros_hbm, out_hbm,
                  si_v, di_v, rows_v, acc_sh, sem):
    cid = lax.axis_index("c")
    sid = lax.axis_index("s")
    wid = sid * _NC + cid
    row0 = sid * _RPS

    @pl.when(sid < 15)
    def _():
        pltpu.sync_copy(zeros_hbm, acc_sh.at[pl.ds(row0, _RPS)])

    @pl.when(sid == 15)
    def _():
        pltpu.sync_copy(zeros_hbm.at[pl.ds(0, _RPS_LAST)],
                        acc_sh.at[pl.ds(row0, _RPS_LAST)])

    plsc.subcore_barrier()

    @pl.loop(wid, _NCH2, step=_NW)
    def _(c):
        base = c * _CH2
        pltpu.sync_copy(src_hbm.at[pl.ds(base, _CH2)], si_v)
        pltpu.sync_copy(dst_hbm.at[pl.ds(base, _CH2)], di_v)
        pltpu.async_copy(g_hbm.at[si_v], rows_v, sem).wait()
        pltpu.sync_copy(rows_v, acc_sh.at[di_v], add=True)

    plsc.subcore_barrier()

    @pl.when(sid < 15)
    def _():
        pltpu.sync_copy(acc_sh.at[pl.ds(row0, _RPS)],
                        out_hbm.at[cid, pl.ds(row0, _RPS)])

    @pl.when(sid == 15)
    def _():
        pltpu.sync_copy(acc_sh.at[pl.ds(row0, _RPS_LAST)],
                        out_hbm.at[cid, pl.ds(row0, _RPS_LAST)])


def _mm_body(x_ref, w_ref, o_ref):
    o_ref[...] = lax.dot_general(
        x_ref[...], w_ref[...], (((1,), (1,)), ((), ())),
        preferred_element_type=jnp.float32)


_tc_matmul = pl.pallas_call(
    _mm_body, out_shape=jax.ShapeDtypeStruct((_N, _D), jnp.float32))


def _prep1_body(h_ref, d0_ref, d1_ref, dis_ref, g_ref):
    deg = d0_ref[...] + d1_ref[...] + 1.0
    dis = lax.rsqrt(deg)
    dis_ref[...] = dis
    g_ref[...] = dis * h_ref[...]


_tc_prep1 = pl.pallas_call(
    _prep1_body,
    out_shape=(jax.ShapeDtypeStruct((_N, 1), jnp.float32),
               jax.ShapeDtypeStruct((_N, _D), jnp.float32)))


def _layer_body(p0_ref, p1_ref, g_ref, dis_ref, b_ref, w_ref, go_ref):
    s = dis_ref[...] * (p0_ref[...] + p1_ref[...] + g_ref[...]) + b_ref[...]
    xr = jnp.maximum(s, 0.0)
    h = lax.dot_general(
        xr, w_ref[...], (((1,), (1,)), ((), ())),
        preferred_element_type=jnp.float32)
    go_ref[...] = dis_ref[...] * h


_tc_layer = pl.pallas_call(
    _layer_body, out_shape=jax.ShapeDtypeStruct((_N, _D), jnp.float32))


def _final_body(p0_ref, p1_ref, g_ref, dis_ref, b_ref, wl_ref, bl_ref, o_ref):
    s = dis_ref[...] * (p0_ref[...] + p1_ref[...] + g_ref[...]) + b_ref[...]
    h = jnp.maximum(s, 0.0)
    pooled = jnp.sum(h, axis=0, keepdims=True) / float(_N)
    o_ref[...] = jnp.sum(pooled * wl_ref[...], axis=1, keepdims=True) + bl_ref[...]


_tc_final = pl.pallas_call(
    _final_body, out_shape=jax.ShapeDtypeStruct((1, 1), jnp.float32))


def kernel(x, edge_index, batch, dropout_rate, W1, b1, W2, b2, W3, b3, Wl, bl):
    src = jnp.concatenate(
        [edge_index[0], jnp.zeros((_EPAD,), edge_index.dtype)])
    dst = jnp.concatenate(
        [edge_index[1], jnp.full((_EPAD,), _N, edge_index.dtype)])
    zerosD = jnp.zeros((_RPS, _D), jnp.float32)
    onesD = jnp.ones((_CHUNK, _D), jnp.float32)
    b1r, b2r, b3r = b1[None, :], b2[None, :], b3[None, :]
    blr = bl[None, :]

    deg_parts = _sc_degree(dst, onesD, zerosD)
    h1 = _tc_matmul(x, W1)
    dis, g1 = _tc_prep1(h1, deg_parts[0, :, 0:1], deg_parts[1, :, 0:1])
    p1 = _sc_aggregate(g1, src, dst, zerosD)
    g2 = _tc_layer(p1[0], p1[1], g1, dis, b1r, W2)
    p2 = _sc_aggregate(g2, src, dst, zerosD)
    g3 = _tc_layer(p2[0], p2[1], g2, dis, b2r, W3)
    p3 = _sc_aggregate(g3, src, dst, zerosD)
    return _tc_final(p3[0], p3[1], g3, dis, b3r, Wl, blr)


# speedup vs baseline: 1.7091x; 1.7091x over previous
import functools

import jax
import jax.numpy as jnp
from jax import lax
from jax.experimental import pallas as pl
from jax.experimental.pallas import tpu as pltpu
from jax.experimental.pallas import tpu_sc as plsc

_N = 10000
_D = 128
_E = 320000
_NC = 2
_NS = 16
_NW = _NC * _NS
_CHUNK = 128
_NCHUNK = _E // _CHUNK
_RPS = 632
_RPS_LAST = _N - 15 * _RPS

_mesh = plsc.VectorSubcoreMesh(core_axis_name="c", subcore_axis_name="s")


@functools.partial(
    pl.kernel,
    mesh=_mesh,
    out_type=jax.ShapeDtypeStruct((_NC, _N, _D), jnp.float32),
    scratch_types=[
        pltpu.VMEM((_CHUNK,), jnp.int32),
        pltpu.VMEM((_CHUNK, _D), jnp.float32),
        pltpu.VMEM_SHARED((_N, _D), jnp.float32),
        pltpu.SemaphoreType.DMA,
    ],
)
def _sc_degree(dst_hbm, ones_hbm, zeros_hbm, out_hbm, di_v, ones_v, acc_sh, sem):
    cid = lax.axis_index("c")
    sid = lax.axis_index("s")
    wid = sid * _NC + cid
    row0 = sid * _RPS

    pltpu.sync_copy(ones_hbm, ones_v)

    @pl.when(sid < 15)
    def _():
        pltpu.sync_copy(zeros_hbm, acc_sh.at[pl.ds(row0, _RPS)])

    @pl.when(sid == 15)
    def _():
        pltpu.sync_copy(zeros_hbm.at[pl.ds(0, _RPS_LAST)],
                        acc_sh.at[pl.ds(row0, _RPS_LAST)])

    plsc.subcore_barrier()

    @pl.loop(wid, _NCHUNK, step=_NW)
    def _(c):
        pltpu.sync_copy(dst_hbm.at[pl.ds(c * _CHUNK, _CHUNK)], di_v)
        pltpu.sync_copy(ones_v, acc_sh.at[di_v], add=True)

    plsc.subcore_barrier()

    @pl.when(sid < 15)
    def _():
        pltpu.sync_copy(acc_sh.at[pl.ds(row0, _RPS)],
                        out_hbm.at[cid, pl.ds(row0, _RPS)])

    @pl.when(sid == 15)
    def _():
        pltpu.sync_copy(acc_sh.at[pl.ds(row0, _RPS_LAST)],
                        out_hbm.at[cid, pl.ds(row0, _RPS_LAST)])


@functools.partial(
    pl.kernel,
    mesh=_mesh,
    out_type=jax.ShapeDtypeStruct((_NC, _N, _D), jnp.float32),
    scratch_types=[
        pltpu.VMEM((_CHUNK,), jnp.int32),
        pltpu.VMEM((_CHUNK,), jnp.int32),
        pltpu.VMEM((_CHUNK, _D), jnp.float32),
        pltpu.VMEM_SHARED((_N, _D), jnp.float32),
        pltpu.SemaphoreType.DMA,
    ],
)
def _sc_aggregate(g_hbm, src_hbm, dst_hbm, zeros_hbm, out_hbm,
                  si_v, di_v, rows_v, acc_sh, sem):
    cid = lax.axis_index("c")
    sid = lax.axis_index("s")
    wid = sid * _NC + cid
    row0 = sid * _RPS

    @pl.when(sid < 15)
    def _():
        pltpu.sync_copy(zeros_hbm, acc_sh.at[pl.ds(row0, _RPS)])

    @pl.when(sid == 15)
    def _():
        pltpu.sync_copy(zeros_hbm.at[pl.ds(0, _RPS_LAST)],
                        acc_sh.at[pl.ds(row0, _RPS_LAST)])

    plsc.subcore_barrier()

    @pl.loop(wid, _NCHUNK, step=_NW)
    def _(c):
        pltpu.sync_copy(src_hbm.at[pl.ds(c * _CHUNK, _CHUNK)], si_v)
        pltpu.sync_copy(dst_hbm.at[pl.ds(c * _CHUNK, _CHUNK)], di_v)
        pltpu.async_copy(g_hbm.at[si_v], rows_v, sem).wait()
        pltpu.sync_copy(rows_v, acc_sh.at[di_v], add=True)

    plsc.subcore_barrier()

    @pl.when(sid < 15)
    def _():
        pltpu.sync_copy(acc_sh.at[pl.ds(row0, _RPS)],
                        out_hbm.at[cid, pl.ds(row0, _RPS)])

    @pl.when(sid == 15)
    def _():
        pltpu.sync_copy(acc_sh.at[pl.ds(row0, _RPS_LAST)],
                        out_hbm.at[cid, pl.ds(row0, _RPS_LAST)])


def _mm_body(x_ref, w_ref, o_ref):
    o_ref[...] = lax.dot_general(
        x_ref[...], w_ref[...], (((1,), (1,)), ((), ())),
        preferred_element_type=jnp.float32)


_tc_matmul = pl.pallas_call(
    _mm_body, out_shape=jax.ShapeDtypeStruct((_N, _D), jnp.float32))


def _prep1_body(h_ref, d0_ref, d1_ref, dis_ref, g_ref):
    deg = d0_ref[...] + d1_ref[...] + 1.0
    dis = lax.rsqrt(deg)
    dis_ref[...] = dis
    g_ref[...] = dis * h_ref[...]


_tc_prep1 = pl.pallas_call(
    _prep1_body,
    out_shape=(jax.ShapeDtypeStruct((_N, 1), jnp.float32),
               jax.ShapeDtypeStruct((_N, _D), jnp.float32)))


def _layer_body(p0_ref, p1_ref, g_ref, dis_ref, b_ref, w_ref, go_ref):
    s = dis_ref[...] * (p0_ref[...] + p1_ref[...] + g_ref[...]) + b_ref[...]
    xr = jnp.maximum(s, 0.0)
    h = lax.dot_general(
        xr, w_ref[...], (((1,), (1,)), ((), ())),
        preferred_element_type=jnp.float32)
    go_ref[...] = dis_ref[...] * h


_tc_layer = pl.pallas_call(
    _layer_body, out_shape=jax.ShapeDtypeStruct((_N, _D), jnp.float32))


def _final_body(p0_ref, p1_ref, g_ref, dis_ref, b_ref, wl_ref, bl_ref, o_ref):
    s = dis_ref[...] * (p0_ref[...] + p1_ref[...] + g_ref[...]) + b_ref[...]
    h = jnp.maximum(s, 0.0)
    pooled = jnp.sum(h, axis=0, keepdims=True) / float(_N)
    o_ref[...] = jnp.sum(pooled * wl_ref[...], axis=1, keepdims=True) + bl_ref[...]


_tc_final = pl.pallas_call(
    _final_body, out_shape=jax.ShapeDtypeStruct((1, 1), jnp.float32))


def kernel(x, edge_index, batch, dropout_rate, W1, b1, W2, b2, W3, b3, Wl, bl):
    src = edge_index[0]
    dst = edge_index[1]
    ones16 = jnp.ones((_CHUNK, _D), jnp.float32)
    zerosD = jnp.zeros((_RPS, _D), jnp.float32)
    b1r, b2r, b3r = b1[None, :], b2[None, :], b3[None, :]
    blr = bl[None, :]

    deg_parts = _sc_degree(dst, ones16, zerosD)
    h1 = _tc_matmul(x, W1)
    dis, g1 = _tc_prep1(h1, deg_parts[0, :, 0:1], deg_parts[1, :, 0:1])
    p1 = _sc_aggregate(g1, src, dst, zerosD)
    g2 = _tc_layer(p1[0], p1[1], g1, dis, b1r, W2)
    p2 = _sc_aggregate(g2, src, dst, zerosD)
    g3 = _tc_layer(p2[0], p2[1], g2, dis, b2r, W3)
    p3 = _sc_aggregate(g3, src, dst, zerosD)
    return _tc_final(p3[0], p3[1], g3, dis, b3r, Wl, blr)


# speedup vs baseline: 2.1003x; 1.2288x over previous
import functools

import jax
import jax.numpy as jnp
from jax import lax
from jax.experimental import pallas as pl
from jax.experimental.pallas import tpu as pltpu
from jax.experimental.pallas import tpu_sc as plsc

_N = 10000
_D = 128
_E = 320000
_NC = 2
_NS = 16
_NW = _NC * _NS
_CHUNK = 128
_NCHUNK = _E // _CHUNK
_CH2 = 256
_NCH2 = _E // _CH2
_RPS = 632
_RPS_LAST = _N - 15 * _RPS

_mesh = plsc.VectorSubcoreMesh(core_axis_name="c", subcore_axis_name="s")


@functools.partial(
    pl.kernel,
    mesh=_mesh,
    out_type=jax.ShapeDtypeStruct((_NC, _N, _D), jnp.float32),
    scratch_types=[
        pltpu.VMEM((_CHUNK,), jnp.int32),
        pltpu.VMEM((_CHUNK, _D), jnp.float32),
        pltpu.VMEM_SHARED((_N, _D), jnp.float32),
        pltpu.SemaphoreType.DMA,
    ],
)
def _sc_degree(dst_hbm, ones_hbm, zeros_hbm, out_hbm, di_v, ones_v, acc_sh, sem):
    cid = lax.axis_index("c")
    sid = lax.axis_index("s")
    wid = sid * _NC + cid
    row0 = sid * _RPS

    pltpu.sync_copy(ones_hbm, ones_v)

    @pl.when(sid < 15)
    def _():
        pltpu.sync_copy(zeros_hbm, acc_sh.at[pl.ds(row0, _RPS)])

    @pl.when(sid == 15)
    def _():
        pltpu.sync_copy(zeros_hbm.at[pl.ds(0, _RPS_LAST)],
                        acc_sh.at[pl.ds(row0, _RPS_LAST)])

    plsc.subcore_barrier()

    @pl.loop(wid, _NCHUNK, step=_NW)
    def _(c):
        pltpu.sync_copy(dst_hbm.at[pl.ds(c * _CHUNK, _CHUNK)], di_v)
        pltpu.sync_copy(ones_v, acc_sh.at[di_v], add=True)

    plsc.subcore_barrier()

    @pl.when(sid < 15)
    def _():
        pltpu.sync_copy(acc_sh.at[pl.ds(row0, _RPS)],
                        out_hbm.at[cid, pl.ds(row0, _RPS)])

    @pl.when(sid == 15)
    def _():
        pltpu.sync_copy(acc_sh.at[pl.ds(row0, _RPS_LAST)],
                        out_hbm.at[cid, pl.ds(row0, _RPS_LAST)])


@functools.partial(
    pl.kernel,
    mesh=_mesh,
    out_type=jax.ShapeDtypeStruct((_NC, _N, _D), jnp.float32),
    scratch_types=[
        pltpu.VMEM((_CH2,), jnp.int32),
        pltpu.VMEM((_CH2,), jnp.int32),
        pltpu.VMEM((_CH2, _D), jnp.float32),
        pltpu.VMEM_SHARED((_N, _D), jnp.float32),
        pltpu.SemaphoreType.DMA,
    ],
)
def _sc_aggregate(g_hbm, src_hbm, dst_hbm, zeros_hbm, out_hbm,
                  si_v, di_v, rows_v, acc_sh, sem):
    cid = lax.axis_index("c")
    sid = lax.axis_index("s")
    wid = sid * _NC + cid
    row0 = sid * _RPS

    @pl.when(sid < 15)
    def _():
        pltpu.sync_copy(zeros_hbm, acc_sh.at[pl.ds(row0, _RPS)])

    @pl.when(sid == 15)
    def _():
        pltpu.sync_copy(zeros_hbm.at[pl.ds(0, _RPS_LAST)],
                        acc_sh.at[pl.ds(row0, _RPS_LAST)])

    plsc.subcore_barrier()

    @pl.loop(wid, _NCH2, step=_NW)
    def _(c):
        pltpu.sync_copy(src_hbm.at[pl.ds(c * _CH2, _CH2)], si_v)
        pltpu.sync_copy(dst_hbm.at[pl.ds(c * _CH2, _CH2)], di_v)
        pltpu.async_copy(g_hbm.at[si_v], rows_v, sem).wait()
        pltpu.sync_copy(rows_v, acc_sh.at[di_v], add=True)

    plsc.subcore_barrier()

    @pl.when(sid < 15)
    def _():
        pltpu.sync_copy(acc_sh.at[pl.ds(row0, _RPS)],
                        out_hbm.at[cid, pl.ds(row0, _RPS)])

    @pl.when(sid == 15)
    def _():
        pltpu.sync_copy(acc_sh.at[pl.ds(row0, _RPS_LAST)],
                        out_hbm.at[cid, pl.ds(row0, _RPS_LAST)])


def _mm_body(x_ref, w_ref, o_ref):
    o_ref[...] = lax.dot_general(
        x_ref[...], w_ref[...], (((1,), (1,)), ((), ())),
        preferred_element_type=jnp.float32)


_tc_matmul = pl.pallas_call(
    _mm_body, out_shape=jax.ShapeDtypeStruct((_N, _D), jnp.float32))


def _prep1_body(h_ref, d0_ref, d1_ref, dis_ref, g_ref):
    deg = d0_ref[...] + d1_ref[...] + 1.0
    dis = lax.rsqrt(deg)
    dis_ref[...] = dis
    g_ref[...] = dis * h_ref[...]


_tc_prep1 = pl.pallas_call(
    _prep1_body,
    out_shape=(jax.ShapeDtypeStruct((_N, 1), jnp.float32),
               jax.ShapeDtypeStruct((_N, _D), jnp.float32)))


def _layer_body(p0_ref, p1_ref, g_ref, dis_ref, b_ref, w_ref, go_ref):
    s = dis_ref[...] * (p0_ref[...] + p1_ref[...] + g_ref[...]) + b_ref[...]
    xr = jnp.maximum(s, 0.0)
    h = lax.dot_general(
        xr, w_ref[...], (((1,), (1,)), ((), ())),
        preferred_element_type=jnp.float32)
    go_ref[...] = dis_ref[...] * h


_tc_layer = pl.pallas_call(
    _layer_body, out_shape=jax.ShapeDtypeStruct((_N, _D), jnp.float32))


def _final_body(p0_ref, p1_ref, g_ref, dis_ref, b_ref, wl_ref, bl_ref, o_ref):
    s = dis_ref[...] * (p0_ref[...] + p1_ref[...] + g_ref[...]) + b_ref[...]
    h = jnp.maximum(s, 0.0)
    pooled = jnp.sum(h, axis=0, keepdims=True) / float(_N)
    o_ref[...] = jnp.sum(pooled * wl_ref[...], axis=1, keepdims=True) + bl_ref[...]


_tc_final = pl.pallas_call(
    _final_body, out_shape=jax.ShapeDtypeStruct((1, 1), jnp.float32))


def kernel(x, edge_index, batch, dropout_rate, W1, b1, W2, b2, W3, b3, Wl, bl):
    src = edge_index[0]
    dst = edge_index[1]
    ones16 = jnp.ones((_CHUNK, _D), jnp.float32)
    zerosD = jnp.zeros((_RPS, _D), jnp.float32)
    b1r, b2r, b3r = b1[None, :], b2[None, :], b3[None, :]
    blr = bl[None, :]

    deg_parts = _sc_degree(dst, ones16, zerosD)
    h1 = _tc_matmul(x, W1)
    dis, g1 = _tc_prep1(h1, deg_parts[0, :, 0:1], deg_parts[1, :, 0:1])
    p1 = _sc_aggregate(g1, src, dst, zerosD)
    g2 = _tc_layer(p1[0], p1[1], g1, dis, b1r, W2)
    p2 = _sc_aggregate(g2, src, dst, zerosD)
    g3 = _tc_layer(p2[0], p2[1], g2, dis, b2r, W3)
    p3 = _sc_aggregate(g3, src, dst, zerosD)
    return _tc_final(p3[0], p3[1], g3, dis, b3r, Wl, blr)


# speedup vs baseline: 2.1943x; 1.0448x over previous
import functools

import jax
import jax.numpy as jnp
from jax import lax
from jax.experimental import pallas as pl
from jax.experimental.pallas import tpu as pltpu
from jax.experimental.pallas import tpu_sc as plsc

_N = 10000
_D = 128
_E = 320000
_NC = 2
_NS = 16
_NW = _NC * _NS
_CHUNK = 128
_NCHUNK = _E // _CHUNK
_CH2 = 320
_NCH2 = _E // _CH2
_RPS = 632
_RPS_LAST = _N - 15 * _RPS

_mesh = plsc.VectorSubcoreMesh(core_axis_name="c", subcore_axis_name="s")


@functools.partial(
    pl.kernel,
    mesh=_mesh,
    out_type=jax.ShapeDtypeStruct((_NC, _N, _D), jnp.float32),
    scratch_types=[
        pltpu.VMEM((_CHUNK,), jnp.int32),
        pltpu.VMEM((_CHUNK, _D), jnp.float32),
        pltpu.VMEM_SHARED((_N, _D), jnp.float32),
        pltpu.SemaphoreType.DMA,
    ],
)
def _sc_degree(dst_hbm, ones_hbm, zeros_hbm, out_hbm, di_v, ones_v, acc_sh, sem):
    cid = lax.axis_index("c")
    sid = lax.axis_index("s")
    wid = sid * _NC + cid
    row0 = sid * _RPS

    pltpu.sync_copy(ones_hbm, ones_v)

    @pl.when(sid < 15)
    def _():
        pltpu.sync_copy(zeros_hbm, acc_sh.at[pl.ds(row0, _RPS)])

    @pl.when(sid == 15)
    def _():
        pltpu.sync_copy(zeros_hbm.at[pl.ds(0, _RPS_LAST)],
                        acc_sh.at[pl.ds(row0, _RPS_LAST)])

    plsc.subcore_barrier()

    @pl.loop(wid, _NCHUNK, step=_NW)
    def _(c):
        pltpu.sync_copy(dst_hbm.at[pl.ds(c * _CHUNK, _CHUNK)], di_v)
        pltpu.sync_copy(ones_v, acc_sh.at[di_v], add=True)

    plsc.subcore_barrier()

    @pl.when(sid < 15)
    def _():
        pltpu.sync_copy(acc_sh.at[pl.ds(row0, _RPS)],
                        out_hbm.at[cid, pl.ds(row0, _RPS)])

    @pl.when(sid == 15)
    def _():
        pltpu.sync_copy(acc_sh.at[pl.ds(row0, _RPS_LAST)],
                        out_hbm.at[cid, pl.ds(row0, _RPS_LAST)])


@functools.partial(
    pl.kernel,
    mesh=_mesh,
    out_type=jax.ShapeDtypeStruct((_NC, _N, _D), jnp.float32),
    scratch_types=[
        pltpu.VMEM((_CH2,), jnp.int32),
        pltpu.VMEM((_CH2,), jnp.int32),
        pltpu.VMEM((_CH2, _D), jnp.float32),
        pltpu.VMEM_SHARED((_N, _D), jnp.float32),
        pltpu.SemaphoreType.DMA,
    ],
)
def _sc_aggregate(g_hbm, src_hbm, dst_hbm, zeros_hbm, out_hbm,
                  si_v, di_v, rows_v, acc_sh, sem):
    cid = lax.axis_index("c")
    sid = lax.axis_index("s")
    wid = sid * _NC + cid
    row0 = sid * _RPS

    @pl.when(sid < 15)
    def _():
        pltpu.sync_copy(zeros_hbm, acc_sh.at[pl.ds(row0, _RPS)])

    @pl.when(sid == 15)
    def _():
        pltpu.sync_copy(zeros_hbm.at[pl.ds(0, _RPS_LAST)],
                        acc_sh.at[pl.ds(row0, _RPS_LAST)])

    plsc.subcore_barrier()

    @pl.loop(wid, _NCH2, step=_NW)
    def _(c):
        pltpu.sync_copy(src_hbm.at[pl.ds(c * _CH2, _CH2)], si_v)
        pltpu.sync_copy(dst_hbm.at[pl.ds(c * _CH2, _CH2)], di_v)
        pltpu.async_copy(g_hbm.at[si_v], rows_v, sem).wait()
        pltpu.sync_copy(rows_v, acc_sh.at[di_v], add=True)

    plsc.subcore_barrier()

    @pl.when(sid < 15)
    def _():
        pltpu.sync_copy(acc_sh.at[pl.ds(row0, _RPS)],
                        out_hbm.at[cid, pl.ds(row0, _RPS)])

    @pl.when(sid == 15)
    def _():
        pltpu.sync_copy(acc_sh.at[pl.ds(row0, _RPS_LAST)],
                        out_hbm.at[cid, pl.ds(row0, _RPS_LAST)])


def _mm_body(x_ref, w_ref, o_ref):
    o_ref[...] = lax.dot_general(
        x_ref[...], w_ref[...], (((1,), (1,)), ((), ())),
        preferred_element_type=jnp.float32)


_tc_matmul = pl.pallas_call(
    _mm_body, out_shape=jax.ShapeDtypeStruct((_N, _D), jnp.float32))


def _prep1_body(h_ref, d0_ref, d1_ref, dis_ref, g_ref):
    deg = d0_ref[...] + d1_ref[...] + 1.0
    dis = lax.rsqrt(deg)
    dis_ref[...] = dis
    g_ref[...] = dis * h_ref[...]


_tc_prep1 = pl.pallas_call(
    _prep1_body,
    out_shape=(jax.ShapeDtypeStruct((_N, 1), jnp.float32),
               jax.ShapeDtypeStruct((_N, _D), jnp.float32)))


def _layer_body(p0_ref, p1_ref, g_ref, dis_ref, b_ref, w_ref, go_ref):
    s = dis_ref[...] * (p0_ref[...] + p1_ref[...] + g_ref[...]) + b_ref[...]
    xr = jnp.maximum(s, 0.0)
    h = lax.dot_general(
        xr, w_ref[...], (((1,), (1,)), ((), ())),
        preferred_element_type=jnp.float32)
    go_ref[...] = dis_ref[...] * h


_tc_layer = pl.pallas_call(
    _layer_body, out_shape=jax.ShapeDtypeStruct((_N, _D), jnp.float32))


def _final_body(p0_ref, p1_ref, g_ref, dis_ref, b_ref, wl_ref, bl_ref, o_ref):
    s = dis_ref[...] * (p0_ref[...] + p1_ref[...] + g_ref[...]) + b_ref[...]
    h = jnp.maximum(s, 0.0)
    pooled = jnp.sum(h, axis=0, keepdims=True) / float(_N)
    o_ref[...] = jnp.sum(pooled * wl_ref[...], axis=1, keepdims=True) + bl_ref[...]


_tc_final = pl.pallas_call(
    _final_body, out_shape=jax.ShapeDtypeStruct((1, 1), jnp.float32))


def kernel(x, edge_index, batch, dropout_rate, W1, b1, W2, b2, W3, b3, Wl, bl):
    src = edge_index[0]
    dst = edge_index[1]
    ones16 = jnp.ones((_CHUNK, _D), jnp.float32)
    zerosD = jnp.zeros((_RPS, _D), jnp.float32)
    b1r, b2r, b3r = b1[None, :], b2[None, :], b3[None, :]
    blr = bl[None, :]

    deg_parts = _sc_degree(dst, ones16, zerosD)
    h1 = _tc_matmul(x, W1)
    dis, g1 = _tc_prep1(h1, deg_parts[0, :, 0:1], deg_parts[1, :, 0:1])
    p1 = _sc_aggregate(g1, src, dst, zerosD)
    g2 = _tc_layer(p1[0], p1[1], g1, dis, b1r, W2)
    p2 = _sc_aggregate(g2, src, dst, zerosD)
    g3 = _tc_layer(p2[0], p2[1], g2, dis, b2r, W3)
    p3 = _sc_aggregate(g3, src, dst, zerosD)
    return _tc_final(p3[0], p3[1], g3, dis, b3r, Wl, blr)


# speedup vs baseline: 2.2442x; 1.0228x over previous
import functools

import jax
import jax.numpy as jnp
from jax import lax
from jax.experimental import pallas as pl
from jax.experimental.pallas import tpu as pltpu
from jax.experimental.pallas import tpu_sc as plsc

_N = 10000
_D = 128
_E = 320000
_NC = 2
_NS = 16
_NW = _NC * _NS
_CHUNK = 128
_NCHUNK = _E // _CHUNK
_CH2 = 320
_NCH2 = _E // _CH2
_RPS = 632
_RPS_LAST = _N - 15 * _RPS

_mesh = plsc.VectorSubcoreMesh(core_axis_name="c", subcore_axis_name="s")


@functools.partial(
    pl.kernel,
    mesh=_mesh,
    out_type=jax.ShapeDtypeStruct((_NC, _N, _D), jnp.float32),
    scratch_types=[
        pltpu.VMEM((_CH2,), jnp.int32),
        pltpu.VMEM((_CH2, _D), jnp.float32),
        pltpu.VMEM_SHARED((_N, _D), jnp.float32),
        pltpu.SemaphoreType.DMA,
    ],
)
def _sc_degree(dst_hbm, ones_hbm, zeros_hbm, out_hbm, di_v, ones_v, acc_sh, sem):
    cid = lax.axis_index("c")
    sid = lax.axis_index("s")
    wid = sid * _NC + cid
    row0 = sid * _RPS

    pltpu.sync_copy(ones_hbm, ones_v)

    @pl.when(sid < 15)
    def _():
        pltpu.sync_copy(zeros_hbm, acc_sh.at[pl.ds(row0, _RPS)])

    @pl.when(sid == 15)
    def _():
        pltpu.sync_copy(zeros_hbm.at[pl.ds(0, _RPS_LAST)],
                        acc_sh.at[pl.ds(row0, _RPS_LAST)])

    plsc.subcore_barrier()

    @pl.loop(wid, _NCH2, step=_NW)
    def _(c):
        pltpu.sync_copy(dst_hbm.at[pl.ds(c * _CH2, _CH2)], di_v)
        pltpu.sync_copy(ones_v, acc_sh.at[di_v], add=True)

    plsc.subcore_barrier()

    @pl.when(sid < 15)
    def _():
        pltpu.sync_copy(acc_sh.at[pl.ds(row0, _RPS)],
                        out_hbm.at[cid, pl.ds(row0, _RPS)])

    @pl.when(sid == 15)
    def _():
        pltpu.sync_copy(acc_sh.at[pl.ds(row0, _RPS_LAST)],
                        out_hbm.at[cid, pl.ds(row0, _RPS_LAST)])


@functools.partial(
    pl.kernel,
    mesh=_mesh,
    out_type=jax.ShapeDtypeStruct((_NC, _N, _D), jnp.float32),
    scratch_types=[
        pltpu.VMEM((_CH2,), jnp.int32),
        pltpu.VMEM((_CH2,), jnp.int32),
        pltpu.VMEM((_CH2, _D), jnp.float32),
        pltpu.VMEM_SHARED((_N, _D), jnp.float32),
        pltpu.SemaphoreType.DMA,
    ],
)
def _sc_aggregate(g_hbm, src_hbm, dst_hbm, zeros_hbm, out_hbm,
                  si_v, di_v, rows_v, acc_sh, sem):
    cid = lax.axis_index("c")
    sid = lax.axis_index("s")
    wid = sid * _NC + cid
    row0 = sid * _RPS

    @pl.when(sid < 15)
    def _():
        pltpu.sync_copy(zeros_hbm, acc_sh.at[pl.ds(row0, _RPS)])

    @pl.when(sid == 15)
    def _():
        pltpu.sync_copy(zeros_hbm.at[pl.ds(0, _RPS_LAST)],
                        acc_sh.at[pl.ds(row0, _RPS_LAST)])

    plsc.subcore_barrier()

    @pl.loop(wid, _NCH2, step=_NW)
    def _(c):
        pltpu.sync_copy(src_hbm.at[pl.ds(c * _CH2, _CH2)], si_v)
        pltpu.sync_copy(dst_hbm.at[pl.ds(c * _CH2, _CH2)], di_v)
        pltpu.async_copy(g_hbm.at[si_v], rows_v, sem).wait()
        pltpu.sync_copy(rows_v, acc_sh.at[di_v], add=True)

    plsc.subcore_barrier()

    @pl.when(sid < 15)
    def _():
        pltpu.sync_copy(acc_sh.at[pl.ds(row0, _RPS)],
                        out_hbm.at[cid, pl.ds(row0, _RPS)])

    @pl.when(sid == 15)
    def _():
        pltpu.sync_copy(acc_sh.at[pl.ds(row0, _RPS_LAST)],
                        out_hbm.at[cid, pl.ds(row0, _RPS_LAST)])


def _mm_body(x_ref, w_ref, o_ref):
    o_ref[...] = lax.dot_general(
        x_ref[...], w_ref[...], (((1,), (1,)), ((), ())),
        preferred_element_type=jnp.float32)


_tc_matmul = pl.pallas_call(
    _mm_body, out_shape=jax.ShapeDtypeStruct((_N, _D), jnp.float32))


def _prep1_body(h_ref, d0_ref, d1_ref, dis_ref, g_ref):
    deg = d0_ref[...] + d1_ref[...] + 1.0
    dis = lax.rsqrt(deg)
    dis_ref[...] = dis
    g_ref[...] = dis * h_ref[...]


_tc_prep1 = pl.pallas_call(
    _prep1_body,
    out_shape=(jax.ShapeDtypeStruct((_N, 1), jnp.float32),
               jax.ShapeDtypeStruct((_N, _D), jnp.float32)))


def _layer_body(p0_ref, p1_ref, g_ref, dis_ref, b_ref, w_ref, go_ref):
    s = dis_ref[...] * (p0_ref[...] + p1_ref[...] + g_ref[...]) + b_ref[...]
    xr = jnp.maximum(s, 0.0)
    h = lax.dot_general(
        xr, w_ref[...], (((1,), (1,)), ((), ())),
        preferred_element_type=jnp.float32)
    go_ref[...] = dis_ref[...] * h


_tc_layer = pl.pallas_call(
    _layer_body, out_shape=jax.ShapeDtypeStruct((_N, _D), jnp.float32))


def _final_body(p0_ref, p1_ref, g_ref, dis_ref, b_ref, wl_ref, bl_ref, o_ref):
    s = dis_ref[...] * (p0_ref[...] + p1_ref[...] + g_ref[...]) + b_ref[...]
    h = jnp.maximum(s, 0.0)
    pooled = jnp.sum(h, axis=0, keepdims=True) / float(_N)
    o_ref[...] = jnp.sum(pooled * wl_ref[...], axis=1, keepdims=True) + bl_ref[...]


_tc_final = pl.pallas_call(
    _final_body, out_shape=jax.ShapeDtypeStruct((1, 1), jnp.float32))


def kernel(x, edge_index, batch, dropout_rate, W1, b1, W2, b2, W3, b3, Wl, bl):
    src = edge_index[0]
    dst = edge_index[1]
    ones16 = jnp.ones((_CH2, _D), jnp.float32)
    zerosD = jnp.zeros((_RPS, _D), jnp.float32)
    b1r, b2r, b3r = b1[None, :], b2[None, :], b3[None, :]
    blr = bl[None, :]

    deg_parts = _sc_degree(dst, ones16, zerosD)
    h1 = _tc_matmul(x, W1)
    dis, g1 = _tc_prep1(h1, deg_parts[0, :, 0:1], deg_parts[1, :, 0:1])
    p1 = _sc_aggregate(g1, src, dst, zerosD)
    g2 = _tc_layer(p1[0], p1[1], g1, dis, b1r, W2)
    p2 = _sc_aggregate(g2, src, dst, zerosD)
    g3 = _tc_layer(p2[0], p2[1], g2, dis, b2r, W3)
    p3 = _sc_aggregate(g3, src, dst, zerosD)
    return _tc_final(p3[0], p3[1], g3, dis, b3r, Wl, blr)
